# Initial kernel scaffold; baseline (speedup 1.0000x reference)
import jax
import jax.numpy as jnp
from jax.experimental import pallas as pl


def kernel(x, W, b):
    raise NotImplementedError("write your pallas kernel here")


# speedup vs baseline: 4.0697x; 4.0697x over previous
import functools
import math

import jax
import jax.numpy as jnp
import numpy as np
from jax.experimental import pallas as pl

B = 4
S = 8192
D = 768
N1 = 64
N2 = 128
F_HALF = S // 2
TOP_K = 10
K_PAD = 16
DT = 128
DTS = 256
TS = 1024


def _tables():
    f1 = np.arange(N1)
    a = np.arange(N1)
    e64 = 2.0 * np.pi * np.outer(f1, a) / N1
    bb = np.arange(N2)
    tw = 2.0 * np.pi * np.outer(f1, bb) / S
    f2 = np.arange(N2)
    e128 = 2.0 * np.pi * np.outer(bb, f2) / N2
    c = np.float32
    return (c(np.cos(e64)), c(np.sin(e64)), c(np.cos(tw)), c(np.sin(tw)),
            c(np.cos(e128)), c(np.sin(e128)))


def _amp_kernel(x_ref, e64c_ref, e64s_ref, twc_ref, tws_ref, fc_ref, fs_ref,
                amp_ref):
    j = pl.program_id(1)
    xb = x_ref[0]
    xr = xb.reshape(N1, N2 * DT)
    zr = jnp.dot(e64c_ref[...], xr, preferred_element_type=jnp.float32, precision=jax.lax.Precision.HIGHEST)
    zi = -jnp.dot(e64s_ref[...], xr, preferred_element_type=jnp.float32, precision=jax.lax.Precision.HIGHEST)
    zr = zr.reshape(N1, N2, DT)
    zi = zi.reshape(N1, N2, DT)
    twc = twc_ref[...][:, :, None]
    tws = tws_ref[...][:, :, None]
    zr2 = zr * twc + zi * tws
    zi2 = zi * twc - zr * tws
    zr2 = jnp.swapaxes(zr2, 1, 2).reshape(N1 * DT, N2)
    zi2 = jnp.swapaxes(zi2, 1, 2).reshape(N1 * DT, N2)
    fc = fc_ref[...]
    fs = fs_ref[...]
    xre = (jnp.dot(zr2, fc, preferred_element_type=jnp.float32, precision=jax.lax.Precision.HIGHEST)
           + jnp.dot(zi2, fs, preferred_element_type=jnp.float32, precision=jax.lax.Precision.HIGHEST))
    xim = (jnp.dot(zi2, fc, preferred_element_type=jnp.float32, precision=jax.lax.Precision.HIGHEST)
           - jnp.dot(zr2, fs, preferred_element_type=jnp.float32, precision=jax.lax.Precision.HIGHEST))
    amp = jnp.sqrt(xre * xre + xim * xim)
    part = amp.reshape(N1, DT, N2).sum(axis=1)

    @pl.when(j == 0)
    def _():
        amp_ref[0] = part

    @pl.when(j > 0)
    def _():
        amp_ref[0] += part


def _sel_kernel(amp_ref, x_ref, cre_ref, cim_ref, f_ref):
    j = pl.program_id(1)
    amp = amp_ref[0]
    f1g = jax.lax.broadcasted_iota(jnp.int32, (N1, N2), 0)
    f2g = jax.lax.broadcasted_iota(jnp.int32, (N1, N2), 1)
    fidx = f1g + N1 * f2g
    dead = jnp.float32(-1.0)
    ampm = jnp.where(fidx <= F_HALF, amp, dead)
    lane = jax.lax.broadcasted_iota(jnp.int32, (1, 128), 1)
    subl = jax.lax.broadcasted_iota(jnp.int32, (K_PAD, 1), 0)

    def body(k, carry):
        am, f_row, f_col = carry
        m = jnp.max(am)
        sel = jnp.min(jnp.where(am == m, fidx, jnp.int32(1 << 30)))
        f_row = jnp.where(lane == k, sel, f_row)
        f_col = jnp.where(subl == k, sel, f_col)
        am = jnp.where(fidx == sel, dead, am)
        return am, f_row, f_col

    _, f_row, f_col = jax.lax.fori_loop(
        0, TOP_K, body,
        (ampm, jnp.zeros((1, 128), jnp.int32), jnp.zeros((K_PAD, 1), jnp.int32)))

    @pl.when(j == 0)
    def _():
        f_ref[0] = f_row

    t_row = jax.lax.broadcasted_iota(jnp.int32, (1, S), 1)
    ph = (f_col * t_row) & (S - 1)
    ang = ph.astype(jnp.float32) * jnp.float32(2.0 * math.pi / S)
    bc = jnp.cos(ang)
    bs = jnp.sin(ang)
    xb = x_ref[0]
    xre = jnp.dot(bc, xb, preferred_element_type=jnp.float32, precision=jax.lax.Precision.HIGHEST)
    xim = -jnp.dot(bs, xb, preferred_element_type=jnp.float32, precision=jax.lax.Precision.HIGHEST)
    valid = subl < TOP_K
    interior = (f_col > 0) & (f_col < F_HALF)
    scale = jnp.where(valid,
                      jnp.where(interior, jnp.float32(2.0 / S),
                                jnp.float32(1.0 / S)),
                      jnp.float32(0.0))
    cre_ref[0] = xre * scale
    cim_ref[0] = xim * (-scale)


def _fold_w_kernel(cre_ref, cim_ref, w_ref, are_ref, aim_ref):
    dn = (((1,), (1,)), ((), ()))
    are_ref[0] = jax.lax.dot_general(cre_ref[0], w_ref[...], dn,
                                     preferred_element_type=jnp.float32, precision=jax.lax.Precision.HIGHEST)
    aim_ref[0] = jax.lax.dot_general(cim_ref[0], w_ref[...], dn,
                                     preferred_element_type=jnp.float32, precision=jax.lax.Precision.HIGHEST)


def _syn_kernel(f_ref, are_ref, aim_ref, bias_ref, out_ref):
    j = pl.program_id(1)
    f_row = f_ref[0][:, :K_PAD]
    t_col = jax.lax.broadcasted_iota(jnp.int32, (TS, 1), 0) + j * TS
    ph = (t_col * f_row) & (S - 1)
    ang = ph.astype(jnp.float32) * jnp.float32(2.0 * math.pi / S)
    bc = jnp.cos(ang)
    bs = jnp.sin(ang)
    out = (jnp.dot(bc, are_ref[0], preferred_element_type=jnp.float32, precision=jax.lax.Precision.HIGHEST)
           + jnp.dot(bs, aim_ref[0], preferred_element_type=jnp.float32, precision=jax.lax.Precision.HIGHEST)
           + bias_ref[...])
    out_ref[0] = out


@jax.jit
def kernel(x, W, b):
    e64c, e64s, twc, tws, fc, fs = _tables()
    tbl = lambda arr: jnp.asarray(arr)

    amp = pl.pallas_call(
        _amp_kernel,
        grid=(B, D // DT),
        in_specs=[
            pl.BlockSpec((1, S, DT), lambda i, j: (i, 0, j)),
            pl.BlockSpec((N1, N1), lambda i, j: (0, 0)),
            pl.BlockSpec((N1, N1), lambda i, j: (0, 0)),
            pl.BlockSpec((N1, N2), lambda i, j: (0, 0)),
            pl.BlockSpec((N1, N2), lambda i, j: (0, 0)),
            pl.BlockSpec((N2, N2), lambda i, j: (0, 0)),
            pl.BlockSpec((N2, N2), lambda i, j: (0, 0)),
        ],
        out_specs=pl.BlockSpec((1, N1, N2), lambda i, j: (i, 0, 0)),
        out_shape=jax.ShapeDtypeStruct((B, N1, N2), jnp.float32),
    )(x, tbl(e64c), tbl(e64s), tbl(twc), tbl(tws), tbl(fc), tbl(fs))

    cre, cim, f_out = pl.pallas_call(
        _sel_kernel,
        grid=(B, D // DTS),
        in_specs=[
            pl.BlockSpec((1, N1, N2), lambda i, j: (i, 0, 0)),
            pl.BlockSpec((1, S, DTS), lambda i, j: (i, 0, j)),
        ],
        out_specs=[
            pl.BlockSpec((1, K_PAD, DTS), lambda i, j: (i, 0, j)),
            pl.BlockSpec((1, K_PAD, DTS), lambda i, j: (i, 0, j)),
            pl.BlockSpec((1, 1, 128), lambda i, j: (i, 0, 0)),
        ],
        out_shape=[
            jax.ShapeDtypeStruct((B, K_PAD, D), jnp.float32),
            jax.ShapeDtypeStruct((B, K_PAD, D), jnp.float32),
            jax.ShapeDtypeStruct((B, 1, 128), jnp.int32),
        ],
    )(amp, x)

    are, aim = pl.pallas_call(
        _fold_w_kernel,
        grid=(B,),
        in_specs=[
            pl.BlockSpec((1, K_PAD, D), lambda i: (i, 0, 0)),
            pl.BlockSpec((1, K_PAD, D), lambda i: (i, 0, 0)),
            pl.BlockSpec((D, D), lambda i: (0, 0)),
        ],
        out_specs=[
            pl.BlockSpec((1, K_PAD, D), lambda i: (i, 0, 0)),
            pl.BlockSpec((1, K_PAD, D), lambda i: (i, 0, 0)),
        ],
        out_shape=[
            jax.ShapeDtypeStruct((B, K_PAD, D), jnp.float32),
            jax.ShapeDtypeStruct((B, K_PAD, D), jnp.float32),
        ],
    )(cre, cim, W)

    out = pl.pallas_call(
        _syn_kernel,
        grid=(B, S // TS),
        in_specs=[
            pl.BlockSpec((1, 1, 128), lambda i, j: (i, 0, 0)),
            pl.BlockSpec((1, K_PAD, D), lambda i, j: (i, 0, 0)),
            pl.BlockSpec((1, K_PAD, D), lambda i, j: (i, 0, 0)),
            pl.BlockSpec((1, D), lambda i, j: (0, 0)),
        ],
        out_specs=pl.BlockSpec((1, TS, D), lambda i, j: (i, j, 0)),
        out_shape=jax.ShapeDtypeStruct((B, S, D), jnp.float32),
    )(f_out, are, aim, b.reshape(1, D))
    return out


# speedup vs baseline: 6.5339x; 1.6055x over previous
import functools
import math

import jax
import jax.numpy as jnp
import numpy as np
from jax.experimental import pallas as pl
from jax.experimental.pallas import tpu as pltpu

B = 4
S = 8192
D = 768
N1 = 64
N2 = 128
F_HALF = S // 2
TOP_K = 10
K_PAD = 16
DT = 128
DTS = 256
TS = 2048


def _tables():
    f1 = np.arange(N1)
    a = np.arange(N1)
    e64 = 2.0 * np.pi * np.outer(f1, a) / N1
    bb = np.arange(N2)
    tw = 2.0 * np.pi * np.outer(f1, bb) / S
    f2 = np.arange(N2)
    e128 = 2.0 * np.pi * np.outer(bb, f2) / N2
    c = np.float32
    e_stack = np.concatenate([np.cos(e64), -np.sin(e64)], axis=0)
    fc, fs = np.cos(e128), np.sin(e128)
    g = np.block([[fc, -fs], [fs, fc]])
    return (c(e_stack), c(np.cos(tw)), c(np.sin(tw)), c(g))


def _amp_kernel(x_ref, es_ref, twc_ref, tws_ref, g_ref, amp_ref):
    j = pl.program_id(1)
    xb = x_ref[0]
    xr = xb.reshape(N1, N2 * DT)
    z = jnp.dot(es_ref[...], xr, preferred_element_type=jnp.float32,
                precision=jax.lax.Precision.HIGHEST)
    zr = z[:N1].reshape(N1, N2, DT)
    zi = z[N1:].reshape(N1, N2, DT)
    twc = twc_ref[...][:, :, None]
    tws = tws_ref[...][:, :, None]
    zr2 = zr * twc + zi * tws
    zi2 = zi * twc - zr * tws
    zr2 = jnp.swapaxes(zr2, 1, 2).reshape(N1 * DT, N2)
    zi2 = jnp.swapaxes(zi2, 1, 2).reshape(N1 * DT, N2)
    zcat = jnp.concatenate([zr2, zi2], axis=1)
    xc = jnp.dot(zcat, g_ref[...], preferred_element_type=jnp.float32,
                 precision=jax.lax.Precision.HIGHEST)
    xre = xc[:, :N2]
    xim = xc[:, N2:]
    amp = jnp.sqrt(xre * xre + xim * xim)
    part = amp.reshape(N1, DT, N2).sum(axis=1)

    @pl.when(j == 0)
    def _():
        amp_ref[0] = part

    @pl.when(j > 0)
    def _():
        amp_ref[0] += part


def _sel_kernel(amp_ref, x_ref, cre_ref, cim_ref, f_ref, bc_ref, bs_ref):
    j = pl.program_id(1)

    @pl.when(j == 0)
    def _():
        amp = amp_ref[0]
        f1g = jax.lax.broadcasted_iota(jnp.int32, (N1, N2), 0)
        f2g = jax.lax.broadcasted_iota(jnp.int32, (N1, N2), 1)
        fidx = f1g + N1 * f2g
        dead = jnp.float32(-1.0)
        ampm = jnp.where(fidx <= F_HALF, amp, dead)
        subl = jax.lax.broadcasted_iota(jnp.int32, (K_PAD, 1), 0)

        def body(k, carry):
            am, f_col = carry
            m = jnp.max(am)
            sel = jnp.min(jnp.where(am == m, fidx, jnp.int32(1 << 30)))
            f_col = jnp.where(subl == k, sel, f_col)
            am = jnp.where(fidx == sel, dead, am)
            return am, f_col

        _, f_col = jax.lax.fori_loop(
            0, TOP_K, body, (ampm, jnp.zeros((K_PAD, 1), jnp.int32)))
        f_ref[0] = jnp.broadcast_to(f_col, (K_PAD, 128))

        valid = subl < TOP_K
        interior = (f_col > 0) & (f_col < F_HALF)
        scale = jnp.where(valid,
                          jnp.where(interior, jnp.float32(2.0 / S),
                                    jnp.float32(1.0 / S)),
                          jnp.float32(0.0))
        t_row = jax.lax.broadcasted_iota(jnp.int32, (1, S), 1)
        ph = (f_col * t_row) & (S - 1)
        ang = ph.astype(jnp.float32) * jnp.float32(2.0 * math.pi / S)
        bc_ref[...] = jnp.cos(ang) * scale
        bs_ref[...] = jnp.sin(ang) * scale

    xb = x_ref[0]
    cre_ref[0] = jnp.dot(bc_ref[...], xb, preferred_element_type=jnp.float32,
                         precision=jax.lax.Precision.HIGHEST)
    cim_ref[0] = jnp.dot(bs_ref[...], xb, preferred_element_type=jnp.float32,
                         precision=jax.lax.Precision.HIGHEST)


def _fold_w_kernel(cre_ref, cim_ref, w_ref, acat_ref):
    ccat = jnp.concatenate([cre_ref[0], cim_ref[0]], axis=0)
    dn = (((1,), (1,)), ((), ()))
    acat_ref[0] = jax.lax.dot_general(ccat, w_ref[...], dn,
                                      preferred_element_type=jnp.float32,
                                      precision=jax.lax.Precision.HIGHEST)


def _syn_kernel(f_ref, a_ref, bias_ref, out_ref):
    j = pl.program_id(1)
    f_col = f_ref[0][:, :1]
    t_row = jax.lax.broadcasted_iota(jnp.int32, (1, TS), 1) + j * TS
    ph = (f_col * t_row) & (S - 1)
    ang = ph.astype(jnp.float32) * jnp.float32(2.0 * math.pi / S)
    bkt = jnp.concatenate([jnp.cos(ang), jnp.sin(ang)], axis=0)
    bas = jnp.swapaxes(bkt, 0, 1)
    out = (jnp.dot(bas, a_ref[0], preferred_element_type=jnp.float32,
                   precision=jax.lax.Precision.HIGHEST)
           + bias_ref[...])
    out_ref[0] = out


@jax.jit
def kernel(x, W, b):
    es, twc, tws, g = _tables()
    tbl = lambda arr: jnp.asarray(arr)

    amp = pl.pallas_call(
        _amp_kernel,
        grid=(B, D // DT),
        in_specs=[
            pl.BlockSpec((1, S, DT), lambda i, j: (i, 0, j)),
            pl.BlockSpec((2 * N1, N1), lambda i, j: (0, 0)),
            pl.BlockSpec((N1, N2), lambda i, j: (0, 0)),
            pl.BlockSpec((N1, N2), lambda i, j: (0, 0)),
            pl.BlockSpec((2 * N2, 2 * N2), lambda i, j: (0, 0)),
        ],
        out_specs=pl.BlockSpec((1, N1, N2), lambda i, j: (i, 0, 0)),
        out_shape=jax.ShapeDtypeStruct((B, N1, N2), jnp.float32),
    )(x, tbl(es), tbl(twc), tbl(tws), tbl(g))

    cre, cim, f_out = pl.pallas_call(
        _sel_kernel,
        grid=(B, D // DTS),
        in_specs=[
            pl.BlockSpec((1, N1, N2), lambda i, j: (i, 0, 0)),
            pl.BlockSpec((1, S, DTS), lambda i, j: (i, 0, j)),
        ],
        out_specs=[
            pl.BlockSpec((1, K_PAD, DTS), lambda i, j: (i, 0, j)),
            pl.BlockSpec((1, K_PAD, DTS), lambda i, j: (i, 0, j)),
            pl.BlockSpec((1, K_PAD, 128), lambda i, j: (i, 0, 0)),
        ],
        out_shape=[
            jax.ShapeDtypeStruct((B, K_PAD, D), jnp.float32),
            jax.ShapeDtypeStruct((B, K_PAD, D), jnp.float32),
            jax.ShapeDtypeStruct((B, K_PAD, 128), jnp.int32),
        ],
        scratch_shapes=[
            pltpu.VMEM((K_PAD, S), jnp.float32),
            pltpu.VMEM((K_PAD, S), jnp.float32),
        ],
    )(amp, x)

    acat = pl.pallas_call(
        _fold_w_kernel,
        grid=(B,),
        in_specs=[
            pl.BlockSpec((1, K_PAD, D), lambda i: (i, 0, 0)),
            pl.BlockSpec((1, K_PAD, D), lambda i: (i, 0, 0)),
            pl.BlockSpec((D, D), lambda i: (0, 0)),
        ],
        out_specs=pl.BlockSpec((1, 2 * K_PAD, D), lambda i: (i, 0, 0)),
        out_shape=jax.ShapeDtypeStruct((B, 2 * K_PAD, D), jnp.float32),
    )(cre, cim, W)

    out = pl.pallas_call(
        _syn_kernel,
        grid=(B, S // TS),
        in_specs=[
            pl.BlockSpec((1, K_PAD, 128), lambda i, j: (i, 0, 0)),
            pl.BlockSpec((1, 2 * K_PAD, D), lambda i, j: (i, 0, 0)),
            pl.BlockSpec((1, D), lambda i, j: (0, 0)),
        ],
        out_specs=pl.BlockSpec((1, TS, D), lambda i, j: (i, j, 0)),
        out_shape=jax.ShapeDtypeStruct((B, S, D), jnp.float32),
    )(f_out, acat, b.reshape(1, D))
    return out


# speedup vs baseline: 6.9149x; 1.0583x over previous
import functools
import math

import jax
import jax.numpy as jnp
import numpy as np
from jax.experimental import pallas as pl
from jax.experimental.pallas import tpu as pltpu

B = 4
S = 8192
D = 768
N1 = 64
N2 = 128
F_HALF = S // 2
TOP_K = 10
K_PAD = 16
DT = 128
DTS = 256
TS = 2048


def _tables():
    f1 = np.arange(N1)
    a = np.arange(N1)
    e64 = 2.0 * np.pi * np.outer(f1, a) / N1
    bb = np.arange(N2)
    tw = 2.0 * np.pi * np.outer(f1, bb) / S
    f2 = np.arange(N2)
    e128 = 2.0 * np.pi * np.outer(bb, f2) / N2
    c = np.float32
    e_stack = np.concatenate([np.cos(e64), -np.sin(e64)], axis=0)
    fc, fs = np.cos(e128), np.sin(e128)
    g = np.block([[fc, -fs], [fs, fc]])
    return (c(e_stack), c(np.cos(tw)), c(np.sin(tw)), c(g))


def _amp_kernel(x_ref, es_ref, twc_ref, tws_ref, g_ref, amp_ref):
    j = pl.program_id(1)
    xb = x_ref[0]
    xr = xb.reshape(N1, N2 * DT)
    z = jnp.dot(es_ref[...], xr, preferred_element_type=jnp.float32,
                precision=jax.lax.Precision.HIGHEST)
    zr = z[:N1].reshape(N1, N2, DT)
    zi = z[N1:].reshape(N1, N2, DT)
    twc = twc_ref[...][:, :, None]
    tws = tws_ref[...][:, :, None]
    zr2 = zr * twc + zi * tws
    zi2 = zi * twc - zr * tws
    zr2 = jnp.swapaxes(zr2, 1, 2).reshape(N1 * DT, N2)
    zi2 = jnp.swapaxes(zi2, 1, 2).reshape(N1 * DT, N2)
    zcat = jnp.concatenate([zr2, zi2], axis=1)
    xc = jnp.dot(zcat, g_ref[...], preferred_element_type=jnp.float32,
                 precision=jax.lax.Precision.HIGHEST)
    xre = xc[:, :N2]
    xim = xc[:, N2:]
    amp = jnp.sqrt(xre * xre + xim * xim)
    part = amp.reshape(N1, DT, N2).sum(axis=1)

    @pl.when(j == 0)
    def _():
        amp_ref[0] = part

    @pl.when(j > 0)
    def _():
        amp_ref[0] += part


def _sel_kernel(amp_ref, x_ref, w_ref, acat_ref, f_ref, bcat_ref, ccat_ref):
    j = pl.program_id(1)

    @pl.when(j == 0)
    def _():
        amp = amp_ref[0]
        f1g = jax.lax.broadcasted_iota(jnp.int32, (N1, N2), 0)
        f2g = jax.lax.broadcasted_iota(jnp.int32, (N1, N2), 1)
        fidx = f1g + N1 * f2g
        dead = jnp.float32(-1.0)
        ampm = jnp.where(fidx <= F_HALF, amp, dead)
        subl = jax.lax.broadcasted_iota(jnp.int32, (K_PAD, 1), 0)

        def body(k, carry):
            am, f_col = carry
            m = jnp.max(am)
            sel = jnp.min(jnp.where(am == m, fidx, jnp.int32(1 << 30)))
            f_col = jnp.where(subl == k, sel, f_col)
            am = jnp.where(fidx == sel, dead, am)
            return am, f_col

        _, f_col = jax.lax.fori_loop(
            0, TOP_K, body, (ampm, jnp.zeros((K_PAD, 1), jnp.int32)))
        f_ref[0] = jnp.broadcast_to(f_col, (K_PAD, 128))

        valid = subl < TOP_K
        interior = (f_col > 0) & (f_col < F_HALF)
        scale = jnp.where(valid,
                          jnp.where(interior, jnp.float32(2.0 / S),
                                    jnp.float32(1.0 / S)),
                          jnp.float32(0.0))
        t_row = jax.lax.broadcasted_iota(jnp.int32, (1, S), 1)
        ph = (f_col * t_row) & (S - 1)
        ang = ph.astype(jnp.float32) * jnp.float32(2.0 * math.pi / S)
        bcat_ref[...] = jnp.concatenate(
            [jnp.cos(ang) * scale, jnp.sin(ang) * scale], axis=0)

    xb = x_ref[0]
    ccat_ref[:, pl.ds(j * DTS, DTS)] = jnp.dot(
        bcat_ref[...], xb, preferred_element_type=jnp.float32,
        precision=jax.lax.Precision.HIGHEST)

    @pl.when(j == D // DTS - 1)
    def _():
        dn = (((1,), (1,)), ((), ()))
        acat_ref[0] = jax.lax.dot_general(ccat_ref[...], w_ref[...], dn,
                                          preferred_element_type=jnp.float32,
                                          precision=jax.lax.Precision.HIGHEST)


def _syn_kernel(f_ref, a_ref, bias_ref, out_ref):
    j = pl.program_id(1)
    f_col = f_ref[0][:, :1]
    t_row = jax.lax.broadcasted_iota(jnp.int32, (1, TS), 1) + j * TS
    ph = (f_col * t_row) & (S - 1)
    ang = ph.astype(jnp.float32) * jnp.float32(2.0 * math.pi / S)
    bkt = jnp.concatenate([jnp.cos(ang), jnp.sin(ang)], axis=0)
    bas = jnp.swapaxes(bkt, 0, 1)
    out = (jnp.dot(bas, a_ref[0], preferred_element_type=jnp.float32,
                   precision=jax.lax.Precision.HIGHEST)
           + bias_ref[...])
    out_ref[0] = out


@jax.jit
def kernel(x, W, b):
    es, twc, tws, g = _tables()
    tbl = lambda arr: jnp.asarray(arr)

    amp = pl.pallas_call(
        _amp_kernel,
        grid=(B, D // DT),
        in_specs=[
            pl.BlockSpec((1, S, DT), lambda i, j: (i, 0, j)),
            pl.BlockSpec((2 * N1, N1), lambda i, j: (0, 0)),
            pl.BlockSpec((N1, N2), lambda i, j: (0, 0)),
            pl.BlockSpec((N1, N2), lambda i, j: (0, 0)),
            pl.BlockSpec((2 * N2, 2 * N2), lambda i, j: (0, 0)),
        ],
        out_specs=pl.BlockSpec((1, N1, N2), lambda i, j: (i, 0, 0)),
        out_shape=jax.ShapeDtypeStruct((B, N1, N2), jnp.float32),
    )(x, tbl(es), tbl(twc), tbl(tws), tbl(g))

    acat, f_out = pl.pallas_call(
        _sel_kernel,
        grid=(B, D // DTS),
        in_specs=[
            pl.BlockSpec((1, N1, N2), lambda i, j: (i, 0, 0)),
            pl.BlockSpec((1, S, DTS), lambda i, j: (i, 0, j)),
            pl.BlockSpec((D, D), lambda i, j: (0, 0)),
        ],
        out_specs=[
            pl.BlockSpec((1, 2 * K_PAD, D), lambda i, j: (i, 0, 0)),
            pl.BlockSpec((1, K_PAD, 128), lambda i, j: (i, 0, 0)),
        ],
        out_shape=[
            jax.ShapeDtypeStruct((B, 2 * K_PAD, D), jnp.float32),
            jax.ShapeDtypeStruct((B, K_PAD, 128), jnp.int32),
        ],
        scratch_shapes=[
            pltpu.VMEM((2 * K_PAD, S), jnp.float32),
            pltpu.VMEM((2 * K_PAD, D), jnp.float32),
        ],
    )(amp, x, W)

    out = pl.pallas_call(
        _syn_kernel,
        grid=(B, S // TS),
        in_specs=[
            pl.BlockSpec((1, K_PAD, 128), lambda i, j: (i, 0, 0)),
            pl.BlockSpec((1, 2 * K_PAD, D), lambda i, j: (i, 0, 0)),
            pl.BlockSpec((1, D), lambda i, j: (0, 0)),
        ],
        out_specs=pl.BlockSpec((1, TS, D), lambda i, j: (i, j, 0)),
        out_shape=jax.ShapeDtypeStruct((B, S, D), jnp.float32),
    )(f_out, acat, b.reshape(1, D))
    return out


# speedup vs baseline: 7.0026x; 1.0127x over previous
import functools
import math

import jax
import jax.numpy as jnp
import numpy as np
from jax.experimental import pallas as pl
from jax.experimental.pallas import tpu as pltpu

B = 4
S = 8192
D = 768
N1 = 64
N2 = 128
F_HALF = S // 2
TOP_K = 10
K_PAD = 16
DT = 128
DTS = 384
TS = 4096


def _tables():
    f1 = np.arange(N1)
    a = np.arange(N1)
    e64 = 2.0 * np.pi * np.outer(f1, a) / N1
    bb = np.arange(N2)
    tw = 2.0 * np.pi * np.outer(f1, bb) / S
    f2 = np.arange(N2)
    e128 = 2.0 * np.pi * np.outer(bb, f2) / N2
    c = np.float32
    e_stack = np.concatenate([np.cos(e64), -np.sin(e64)], axis=0)
    fc, fs = np.cos(e128), np.sin(e128)
    g = np.block([[fc, -fs], [fs, fc]])
    return (c(e_stack), c(np.cos(tw)), c(np.sin(tw)), c(g))


def _amp_kernel(x_ref, es_ref, twc_ref, tws_ref, g_ref, amp_ref):
    j = pl.program_id(1)
    xb = x_ref[0]
    xr = xb.reshape(N1, N2 * DT)
    z = jnp.dot(es_ref[...], xr, preferred_element_type=jnp.float32,
                precision=jax.lax.Precision.HIGHEST)
    zr = z[:N1].reshape(N1, N2, DT)
    zi = z[N1:].reshape(N1, N2, DT)
    twc = twc_ref[...][:, :, None]
    tws = tws_ref[...][:, :, None]
    zr2 = zr * twc + zi * tws
    zi2 = zi * twc - zr * tws
    zr2 = jnp.swapaxes(zr2, 1, 2).reshape(N1 * DT, N2)
    zi2 = jnp.swapaxes(zi2, 1, 2).reshape(N1 * DT, N2)
    zcat = jnp.concatenate([zr2, zi2], axis=1)
    xc = jnp.dot(zcat, g_ref[...], preferred_element_type=jnp.float32,
                 precision=jax.lax.Precision.HIGHEST)
    xre = xc[:, :N2]
    xim = xc[:, N2:]
    amp = jnp.sqrt(xre * xre + xim * xim)
    part = amp.reshape(N1, DT, N2).sum(axis=1)

    @pl.when(j == 0)
    def _():
        amp_ref[0] = part

    @pl.when(j > 0)
    def _():
        amp_ref[0] += part


def _sel_kernel(amp_ref, x_ref, w_ref, acat_ref, f_ref, bcat_ref, ccat_ref):
    j = pl.program_id(1)

    @pl.when(j == 0)
    def _():
        amp = amp_ref[0]
        f1g = jax.lax.broadcasted_iota(jnp.int32, (N1, N2), 0)
        f2g = jax.lax.broadcasted_iota(jnp.int32, (N1, N2), 1)
        fidx = f1g + N1 * f2g
        dead = jnp.float32(-1.0)
        ampm = jnp.where(fidx <= F_HALF, amp, dead)
        subl = jax.lax.broadcasted_iota(jnp.int32, (K_PAD, 1), 0)

        def body(k, carry):
            am, f_col = carry
            m = jnp.max(am)
            sel = jnp.min(jnp.where(am == m, fidx, jnp.int32(1 << 30)))
            f_col = jnp.where(subl == k, sel, f_col)
            am = jnp.where(fidx == sel, dead, am)
            return am, f_col

        _, f_col = jax.lax.fori_loop(
            0, TOP_K, body, (ampm, jnp.zeros((K_PAD, 1), jnp.int32)))
        f_ref[0] = jnp.broadcast_to(f_col, (K_PAD, 128))

        valid = subl < TOP_K
        interior = (f_col > 0) & (f_col < F_HALF)
        scale = jnp.where(valid,
                          jnp.where(interior, jnp.float32(2.0 / S),
                                    jnp.float32(1.0 / S)),
                          jnp.float32(0.0))
        t_row = jax.lax.broadcasted_iota(jnp.int32, (1, S), 1)
        ph = (f_col * t_row) & (S - 1)
        ang = ph.astype(jnp.float32) * jnp.float32(2.0 * math.pi / S)
        bcat_ref[...] = jnp.concatenate(
            [jnp.cos(ang) * scale, jnp.sin(ang) * scale], axis=0)

    xb = x_ref[0]
    ccat_ref[:, pl.ds(j * DTS, DTS)] = jnp.dot(
        bcat_ref[...], xb, preferred_element_type=jnp.float32,
        precision=jax.lax.Precision.HIGHEST)

    @pl.when(j == D // DTS - 1)
    def _():
        dn = (((1,), (1,)), ((), ()))
        acat_ref[0] = jax.lax.dot_general(ccat_ref[...], w_ref[...], dn,
                                          preferred_element_type=jnp.float32,
                                          precision=jax.lax.Precision.HIGHEST)


def _syn_kernel(f_ref, a_ref, bias_ref, out_ref):
    j = pl.program_id(1)
    f_col = f_ref[0][:, :1]
    t_row = jax.lax.broadcasted_iota(jnp.int32, (1, TS), 1) + j * TS
    ph = (f_col * t_row) & (S - 1)
    ang = ph.astype(jnp.float32) * jnp.float32(2.0 * math.pi / S)
    bkt = jnp.concatenate([jnp.cos(ang), jnp.sin(ang)], axis=0)
    bas = jnp.swapaxes(bkt, 0, 1)
    out = (jnp.dot(bas, a_ref[0], preferred_element_type=jnp.float32,
                   precision=jax.lax.Precision.HIGHEST)
           + bias_ref[...])
    out_ref[0] = out


@jax.jit
def kernel(x, W, b):
    es, twc, tws, g = _tables()
    tbl = lambda arr: jnp.asarray(arr)

    amp = pl.pallas_call(
        _amp_kernel,
        grid=(B, D // DT),
        in_specs=[
            pl.BlockSpec((1, S, DT), lambda i, j: (i, 0, j)),
            pl.BlockSpec((2 * N1, N1), lambda i, j: (0, 0)),
            pl.BlockSpec((N1, N2), lambda i, j: (0, 0)),
            pl.BlockSpec((N1, N2), lambda i, j: (0, 0)),
            pl.BlockSpec((2 * N2, 2 * N2), lambda i, j: (0, 0)),
        ],
        out_specs=pl.BlockSpec((1, N1, N2), lambda i, j: (i, 0, 0)),
        out_shape=jax.ShapeDtypeStruct((B, N1, N2), jnp.float32),
    )(x, tbl(es), tbl(twc), tbl(tws), tbl(g))

    acat, f_out = pl.pallas_call(
        _sel_kernel,
        grid=(B, D // DTS),
        in_specs=[
            pl.BlockSpec((1, N1, N2), lambda i, j: (i, 0, 0)),
            pl.BlockSpec((1, S, DTS), lambda i, j: (i, 0, j)),
            pl.BlockSpec((D, D), lambda i, j: (0, 0)),
        ],
        out_specs=[
            pl.BlockSpec((1, 2 * K_PAD, D), lambda i, j: (i, 0, 0)),
            pl.BlockSpec((1, K_PAD, 128), lambda i, j: (i, 0, 0)),
        ],
        out_shape=[
            jax.ShapeDtypeStruct((B, 2 * K_PAD, D), jnp.float32),
            jax.ShapeDtypeStruct((B, K_PAD, 128), jnp.int32),
        ],
        scratch_shapes=[
            pltpu.VMEM((2 * K_PAD, S), jnp.float32),
            pltpu.VMEM((2 * K_PAD, D), jnp.float32),
        ],
    )(amp, x, W)

    out = pl.pallas_call(
        _syn_kernel,
        grid=(B, S // TS),
        in_specs=[
            pl.BlockSpec((1, K_PAD, 128), lambda i, j: (i, 0, 0)),
            pl.BlockSpec((1, 2 * K_PAD, D), lambda i, j: (i, 0, 0)),
            pl.BlockSpec((1, D), lambda i, j: (0, 0)),
        ],
        out_specs=pl.BlockSpec((1, TS, D), lambda i, j: (i, j, 0)),
        out_shape=jax.ShapeDtypeStruct((B, S, D), jnp.float32),
    )(f_out, acat, b.reshape(1, D))
    return out
